# baseline (speedup 1.0000x reference)
import functools
import itertools

import jax
import jax.numpy as jnp
from jax.experimental import pallas as pl
from jax.experimental.pallas import tpu as pltpu

_SPECS = [
    (64, 16, (3, 3, 3), (1, 1, 1)),
    (16, 16, (3, 3, 3), (1, 1, 1)),
    (16, 32, (3, 3, 3), (2, 2, 2)),
    (32, 32, (3, 3, 3), (1, 1, 1)),
    (32, 64, (3, 3, 3), (2, 2, 2)),
    (64, 64, (3, 3, 3), (1, 1, 1)),
    (64, 64, (3, 3, 3), (2, 2, 2)),
    (64, 64, (3, 3, 3), (1, 1, 1)),
    (64, 64, (3, 1, 1), (2, 1, 1)),
]
_B = 4
_G0 = (16, 48, 48)
_EPS = 1e-5

_BK_BY_S = {180000: 3600, 27040: 1352, 4704: 1176, 1024: 1024}


def _make_plans():
    plans = []
    g = _G0
    for (ci, co, k, s) in _SPECS:
        D, H, W = g
        Hp, Wp = H + 2, W + 2
        pads = tuple((kk - 1) // 2 for kk in k)
        offs = [
            (kd - pads[0]) * Hp * Wp + (kh - pads[1]) * Wp + (kw - pads[2])
            for kd in range(k[0])
            for kh in range(k[1])
            for kw in range(k[2])
        ]
        m = max(max(offs), -min(offs))
        S = _B * (D + 2) * Hp * Wp
        Bk = _BK_BY_S[S]
        gout = tuple(gg // ss for gg, ss in zip(g, s))
        plans.append(dict(g=g, gout=gout, offs=tuple(offs), m=m, S=S, Bk=Bk,
                          s=s, strided=any(ss > 1 for ss in s)))
        g = gout
    return plans


_PLANS = _make_plans()


def _conv_body(xp, xc, xn, mp, mc, mn, sel, w, sc, sh,
               y, p1, p2, pc, *, Bk, m, offs, norm, Cout):
    window = jnp.concatenate([xp[...], xc[...], xn[...]], axis=0)
    need = window[Bk - m:2 * Bk + m, :]
    if norm:
        mwin = jnp.concatenate([mp[...], mc[...], mn[...]], axis=0)
        mwin = mwin[Bk - m:2 * Bk + m, :]
        need = jnp.maximum(need * sc[...] + sh[...], 0.0) * mwin
    acc = jnp.zeros((Bk, Cout), jnp.float32)
    for t, off in enumerate(offs):
        sl = need[m + off:m + off + Bk, :]
        acc = acc + jax.lax.dot_general(
            sl, w[t], (((1,), (0,)), ((), ())),
            preferred_element_type=jnp.float32)
    y[...] = acc
    s = sel[...]
    p1[...] = jnp.sum(acc * s, axis=0).reshape(1, 1, Cout)
    p2[...] = jnp.sum(acc * acc * s, axis=0).reshape(1, 1, Cout)
    pc[...] = jnp.full((1, 1, 8), jnp.sum(s), jnp.float32)


def _conv_layer(x, mask, sel, w_taps, scale, shift, plan, norm):
    S, Cin = x.shape
    T, _, Cout = w_taps.shape
    Bk, m, offs = plan['Bk'], plan['m'], plan['offs']
    Nb = S // Bk

    def ip(i):
        return (jnp.maximum(i - 1, 0), 0)

    def ic(i):
        return (i, 0)

    def inx(i):
        return (jnp.minimum(i + 1, Nb - 1), 0)

    body = functools.partial(_conv_body, Bk=Bk, m=m, offs=offs, norm=norm,
                             Cout=Cout)
    y, p1, p2, pc = pl.pallas_call(
        body,
        grid=(Nb,),
        in_specs=[
            pl.BlockSpec((Bk, Cin), ip),
            pl.BlockSpec((Bk, Cin), ic),
            pl.BlockSpec((Bk, Cin), inx),
            pl.BlockSpec((Bk, 1), ip),
            pl.BlockSpec((Bk, 1), ic),
            pl.BlockSpec((Bk, 1), inx),
            pl.BlockSpec((Bk, 1), ic),
            pl.BlockSpec((T, Cin, Cout), lambda i: (0, 0, 0)),
            pl.BlockSpec((1, Cin), lambda i: (0, 0)),
            pl.BlockSpec((1, Cin), lambda i: (0, 0)),
        ],
        out_specs=[
            pl.BlockSpec((Bk, Cout), ic),
            pl.BlockSpec((1, 1, Cout), lambda i: (i, 0, 0)),
            pl.BlockSpec((1, 1, Cout), lambda i: (i, 0, 0)),
            pl.BlockSpec((1, 1, 8), lambda i: (i, 0, 0)),
        ],
        out_shape=[
            jax.ShapeDtypeStruct((S, Cout), jnp.float32),
            jax.ShapeDtypeStruct((Nb, 1, Cout), jnp.float32),
            jax.ShapeDtypeStruct((Nb, 1, Cout), jnp.float32),
            jax.ShapeDtypeStruct((Nb, 1, 8), jnp.float32),
        ],
    )(x, x, x, mask, mask, mask, sel, w_taps, scale, shift)
    return y, p1.sum(axis=(0, 1)), p2.sum(axis=(0, 1)), pc[:, 0, 0].sum()


def _norm_body(yr, mr, scr, shr, orf):
    orf[...] = jnp.maximum(yr[...] * scr[...] + shr[...], 0.0) * mr[...]


def _pool_mask(mflat, g, s):
    D, H, W = g
    m5 = mflat.reshape(_B, D + 2, H + 2, W + 2)[:, 1:1 + D, 1:1 + H, 1:1 + W]
    m5 = m5.reshape(_B, D // s[0], s[0], H // s[1], s[1], W // s[2], s[2])
    return m5.max(axis=(2, 4, 6))


def _embed_sel(pooled, g, s):
    D, H, W = g
    z = jnp.zeros((_B, D + 2, H + 2, W + 2), jnp.float32)
    z = z.at[:, 1:1 + D:s[0], 1:1 + H:s[1], 1:1 + W:s[2]].set(pooled)
    return z.reshape(-1, 1)


def _subsample(yflat, g, s):
    D, H, W = g
    C = yflat.shape[-1]
    y5 = yflat.reshape(_B, D + 2, H + 2, W + 2, C)
    return y5[:, 1:1 + D:s[0], 1:1 + H:s[1], 1:1 + W:s[2], :]


def _pad_flat(x5):
    C = x5.shape[-1]
    xp = jnp.pad(x5, ((0, 0), (1, 1), (1, 1), (1, 1), (0, 0)))
    return xp.reshape(-1, C)


def kernel(voxel_features, coors, batch_size, input_shape, params):
    del batch_size, input_shape
    D0, H0, W0 = _G0
    Dp, Hp, Wp = D0 + 2, H0 + 2, W0 + 2
    b, d, h, w = coors[:, 0], coors[:, 1], coors[:, 2], coors[:, 3]
    r = ((b * Dp + (d + 1)) * Hp + (h + 1)) * Wp + (w + 1)
    S0 = _B * Dp * Hp * Wp
    x = jnp.zeros((S0, 64), jnp.float32).at[r].add(voxel_features)
    mask = jnp.zeros((S0, 1), jnp.float32).at[r].set(1.0)

    Cin0 = 64
    scale = jnp.ones((1, Cin0), jnp.float32)
    shift = jnp.zeros((1, Cin0), jnp.float32)

    for li, (spec, plan) in enumerate(zip(_SPECS, _PLANS)):
        ci, co, k, s = spec
        T = len(plan['offs'])
        w_taps = jnp.transpose(params[li]['W'], (2, 3, 4, 1, 0)).reshape(T, ci, co)
        if plan['strided']:
            pooled = _pool_mask(mask, plan['g'], s)
            sel = _embed_sel(pooled, plan['g'], s)
        else:
            pooled = None
            sel = mask
        y, p1, p2, pcnt = _conv_layer(x, mask, sel, w_taps, scale, shift,
                                      plan, norm=(li > 0))
        cnt = jnp.maximum(pcnt, 1.0)
        mu = p1 / cnt
        var = p2 / cnt - mu * mu
        inv = jax.lax.rsqrt(var + _EPS)
        scale = (params[li]['gamma'] * inv).reshape(1, co)
        shift = (params[li]['beta'] - mu * params[li]['gamma'] * inv).reshape(1, co)
        if plan['strided']:
            x = _pad_flat(_subsample(y, plan['g'], s))
            mask = _pad_flat(pooled[..., None])
        else:
            x = y

    Sf, Cf = x.shape
    out = pl.pallas_call(
        _norm_body,
        out_shape=jax.ShapeDtypeStruct((Sf, Cf), jnp.float32),
    )(x, mask, scale, shift)

    Df, Hf, Wf = _PLANS[-1]['gout']
    o5 = out.reshape(_B, Df + 2, Hf + 2, Wf + 2, Cf)
    o5 = o5[:, 1:1 + Df, 1:1 + Hf, 1:1 + Wf, :]
    return jnp.transpose(o5, (0, 4, 1, 2, 3))
